# baseline (speedup 1.0000x reference)
import jax
import jax.numpy as jnp
import numpy as np
from jax import lax
from jax.experimental import pallas as pl
from jax.experimental.pallas import tpu as pltpu
from jax.experimental.pallas import tpu_sc as plsc

_ALPHA = 0.0001
_MARGIN = 100.0
_K = 3
_L = 16
_B = 8
_T = 2048
_F = 2048
_FH = _F // 2
_NCHUNK = _T // _L
_NROW = 2 * _B

def _ln(x, lanes):
    bits = plsc.bitcast(x, jnp.int32)
    e = ((bits >> 23) - 127).astype(jnp.float32)
    m = plsc.bitcast((bits & 0x007FFFFF) | 0x3F800000, jnp.float32)
    t = (m - 1.0) / (m + 1.0)
    t2 = t * t
    p = 2.0 + t2 * (0.6666666 + t2 * (0.4 + t2 * (0.28571429 + t2 * 0.22222222)))
    ln = e * 0.69314718 + t * p
    return jnp.where(x < 1e-37, -1e4, ln)


def _sqrt(x):
    bits = plsc.bitcast(x, jnp.int32)
    y = plsc.bitcast(0x5F3759DF - (bits >> 1), jnp.float32)
    for _ in range(3):
        y = y * (1.5 - 0.5 * x * y * y)
    return x * y


def _sc_body(fmagn_a, fmagn_n, sls_a, sls_n, tab_a, tab_n, ld_hbm,
             out_hbm, stage_hbm,
             fm_v, sl_v, rows_v, res_v, tmp_v, tmp2_v, ld_v, cmb_v,
             sem, sem2, sem3):
    w = lax.axis_index("s")
    core = lax.axis_index("c")
    lanes = lax.iota(jnp.int32, _L)

    @pl.when(core == 0)
    def _core0():
        _row_work(fmagn_a, fmagn_n, sls_a, sls_n, tab_a, tab_n,
                  stage_hbm, fm_v, sl_v, rows_v, res_v, sem, sem2, sem3,
                  w, lanes)
        plsc.subcore_barrier()

        @pl.when(w == 0)
        def _final():
            _combine(ld_hbm, out_hbm, stage_hbm, res_v, tmp_v, tmp2_v, ld_v,
                     cmb_v, lanes)


def _row_work(fmagn_a, fmagn_n, sls_a, sls_n, tab_a, tab_n,
              stage_hbm, fm_v, sl_v, rows_v, res_v, sem, sem2, sem3,
              w, lanes):
    @pl.when(w < _B)
    def _():
        pltpu.async_copy(fmagn_a.at[w], fm_v, sem)
        pltpu.async_copy(sls_a.at[w], sl_v, sem2)

    @pl.when(w >= _B)
    def _():
        pltpu.async_copy(fmagn_n.at[w - _B], fm_v, sem)
        pltpu.async_copy(sls_n.at[w - _B], sl_v, sem2)

    pltpu.make_async_copy(fmagn_a.at[0], fm_v, sem).wait()

    neg_inf = jnp.full((_L,), -jnp.inf, jnp.float32)
    zero_i = jnp.zeros((_L,), jnp.int32)

    def insert(stack, v, ix):
        t1, i1, t2, i2, t3, i3 = stack
        c1 = v > t1
        nt1 = jnp.where(c1, v, t1)
        ni1 = jnp.where(c1, ix, i1)
        dv = jnp.where(c1, t1, v)
        di = jnp.where(c1, i1, ix)
        c2 = dv > t2
        nt2 = jnp.where(c2, dv, t2)
        ni2 = jnp.where(c2, di, i2)
        dv2 = jnp.where(c2, t2, dv)
        di2 = jnp.where(c2, i2, di)
        c3 = dv2 > t3
        nt3 = jnp.where(c3, dv2, t3)
        ni3 = jnp.where(c3, di2, i3)
        return (nt1, ni1, nt2, ni2, nt3, ni3)

    def insert_lex(stack, v, ix):
        t1, i1, t2, i2, t3, i3 = stack
        c1 = (v > t1) | ((v == t1) & (ix < i1))
        nt1 = jnp.where(c1, v, t1)
        ni1 = jnp.where(c1, ix, i1)
        dv = jnp.where(c1, t1, v)
        di = jnp.where(c1, i1, ix)
        c2 = (dv > t2) | ((dv == t2) & (di < i2))
        nt2 = jnp.where(c2, dv, t2)
        ni2 = jnp.where(c2, di, i2)
        dv2 = jnp.where(c2, t2, dv)
        di2 = jnp.where(c2, i2, di)
        c3 = (dv2 > t3) | ((dv2 == t3) & (di2 < i3))
        nt3 = jnp.where(c3, dv2, t3)
        ni3 = jnp.where(c3, di2, i3)
        return (nt1, ni1, nt2, ni2, nt3, ni3)

    empty = (neg_inf, zero_i, neg_inf, zero_i, neg_inf, zero_i)

    def topk_step(j, carry):
        sa = carry[:6]
        sb = carry[6:]
        va = fm_v[pl.ds((2 * j) * _L, _L)]
        vb = fm_v[pl.ds((2 * j + 1) * _L, _L)]
        ixa = lanes + (2 * j) * _L
        ixb = lanes + (2 * j + 1) * _L
        return insert(sa, va, ixa) + insert(sb, vb, ixb)

    st = lax.fori_loop(0, _NCHUNK // 2, topk_step, empty + empty)
    t1, i1, t2, i2, t3, i3 = st[:6]
    sb = st[6:]
    for lvl in range(3):
        t1, i1, t2, i2, t3, i3 = insert_lex(
            (t1, i1, t2, i2, t3, i3), sb[2 * lvl], sb[2 * lvl + 1])

    sel = []
    for _r in range(_K):
        m = jnp.max(t1)
        s = jnp.min(jnp.where(t1 == m, i1, _T))
        rm = i1 == s
        t1 = jnp.where(rm, t2, t1)
        i1 = jnp.where(rm, i2, i1)
        t2 = jnp.where(rm, t3, t2)
        i2 = jnp.where(rm, i3, i2)
        t3 = jnp.where(rm, neg_inf, t3)
        sel.append(s)

    b = jnp.where(w < _B, w, w - _B)
    selv = jnp.where(lanes == 1, sel[1],
                     jnp.where(lanes == 2, sel[2], sel[0]))
    row_iv = b * _T + selv

    @pl.when(w < _B)
    def _():
        pltpu.async_copy(tab_a.at[row_iv], rows_v, sem)

    @pl.when(w >= _B)
    def _():
        pltpu.async_copy(tab_n.at[row_iv], rows_v, sem)

    pltpu.make_async_copy(sls_a.at[0], sl_v, sem2).wait()
    g = plsc.load_gather(sl_v, [selv])
    vls = jnp.sum(jnp.where(lanes < _K, g, 0.0)) * (1.0 / _K)

    pltpu.make_async_copy(tab_a.at[row_iv], rows_v, sem).wait()

    def ssq_step(c, accs):
        acc0, acc1 = accs
        d0 = pl.ds((2 * c) * _L, _L)
        d1 = pl.ds((2 * c + 1) * _L, _L)
        a0 = (rows_v[0, d0] + rows_v[1, d0] + rows_v[2, d0]) * (1.0 / 3.0)
        a1 = (rows_v[0, d1] + rows_v[1, d1] + rows_v[2, d1]) * (1.0 / 3.0)
        return (acc0 + a0 * a0, acc1 + a1 * a1)

    z = jnp.zeros((_L,), jnp.float32)
    acc0, acc1 = lax.fori_loop(0, _F // (2 * _L), ssq_step, (z, z))
    ssq = jnp.sum(acc0 + acc1)

    res_v[...] = jnp.where(lanes == 0, ssq, jnp.where(lanes == 1, vls, 0.0))
    pltpu.sync_copy(res_v, stage_hbm.at[w])


def _combine(ld_hbm, out_hbm, stage_hbm, res_v, tmp_v, tmp2_v, ld_v,
             cmb_v, lanes):
    if True:
        pltpu.sync_copy(stage_hbm, cmb_v)
        pltpu.sync_copy(ld_hbm, ld_v)
        zeros = jnp.zeros((_L,), jnp.int32)
        ssqv = plsc.load_gather(cmb_v, [lanes, zeros])
        vlsv = plsc.load_gather(cmb_v, [lanes, zeros + 1])

        sq = _sqrt(ssqv)
        tmp_v[...] = sq
        tmp2_v[...] = vlsv
        rot = (lanes + _B) & (_L - 1)
        sq_shift = plsc.load_gather(tmp_v, [rot])
        vc = plsc.load_gather(tmp2_v, [rot])

        la = jnp.abs(_MARGIN - sq)
        terms = (la + sq_shift) * (la + sq_shift)
        loss_rtfm = jnp.sum(jnp.where(lanes < _B, terms, 0.0)) * (1.0 / _B)

        ld = ld_v[...]
        ln_p = jnp.maximum(_ln(vc, lanes), -100.0)
        ln_1mp = jnp.maximum(_ln(1.0 - vc, lanes), -100.0)
        bce = -(ld * ln_p + (1.0 - ld) * ln_1mp)
        loss_vls = jnp.sum(bce) * (1.0 / _NROW)

        res_v[...] = jnp.where(lanes == 0, _ALPHA * loss_rtfm,
                               jnp.where(lanes == 1, loss_vls, 0.0))
        pltpu.sync_copy(res_v, out_hbm)


_sc_call = pl.kernel(
    _sc_body,
    out_type=[jax.ShapeDtypeStruct((_L,), jnp.float32),
              jax.ShapeDtypeStruct((_NROW, _L), jnp.float32)],
    mesh=plsc.VectorSubcoreMesh(core_axis_name="c", subcore_axis_name="s"),
    scratch_types=[
        pltpu.VMEM((_T,), jnp.float32),
        pltpu.VMEM((_T,), jnp.float32),
        pltpu.VMEM((_L, _F), jnp.float32),
        pltpu.VMEM((_L,), jnp.float32),
        pltpu.VMEM((_L,), jnp.float32),
        pltpu.VMEM((_L,), jnp.float32),
        pltpu.VMEM((_L,), jnp.float32),
        pltpu.VMEM((_NROW, _L), jnp.float32),
        pltpu.SemaphoreType.DMA,
        pltpu.SemaphoreType.DMA,
        pltpu.SemaphoreType.DMA,
    ],
    compiler_params=pltpu.CompilerParams(needs_layout_passes=False),
)


def kernel(abnr_fmagn, norm_fmagn, abnr_feats, norm_feats, abnr_sls, norm_sls, ldata):
    tab_a = abnr_feats.reshape(_B * _T, _F)
    tab_n = norm_feats.reshape(_B * _T, _F)
    out, _stage = _sc_call(abnr_fmagn, norm_fmagn, abnr_sls, norm_sls,
                           tab_a, tab_n, ldata)
    return (out[0], out[1])


# speedup vs baseline: 1.0639x; 1.0639x over previous
import jax
import jax.numpy as jnp
import numpy as np
from jax import lax
from jax.experimental import pallas as pl
from jax.experimental.pallas import tpu as pltpu
from jax.experimental.pallas import tpu_sc as plsc

_ALPHA = 0.0001
_MARGIN = 100.0
_K = 3
_L = 16
_B = 8
_T = 2048
_F = 2048
_FH = _F // 2
_NCHUNK = _T // _L
_NROW = 2 * _B

def _ln(x, lanes):
    bits = plsc.bitcast(x, jnp.int32)
    e = ((bits >> 23) - 127).astype(jnp.float32)
    m = plsc.bitcast((bits & 0x007FFFFF) | 0x3F800000, jnp.float32)
    t = (m - 1.0) / (m + 1.0)
    t2 = t * t
    p = 2.0 + t2 * (0.6666666 + t2 * (0.4 + t2 * (0.28571429 + t2 * 0.22222222)))
    ln = e * 0.69314718 + t * p
    return jnp.where(x < 1e-37, -1e4, ln)


def _sqrt(x):
    bits = plsc.bitcast(x, jnp.int32)
    y = plsc.bitcast(0x5F3759DF - (bits >> 1), jnp.float32)
    for _ in range(3):
        y = y * (1.5 - 0.5 * x * y * y)
    return x * y


def _sc_body(fmagn_a, fmagn_n, sls_a, sls_n, tab_a, tab_n, ld_hbm,
             out_hbm, stage_hbm,
             fm_v, sl_v, rows_v, res_v, tmp_v, tmp2_v, ld_v, cmb_v,
             sem, sem2):
    w = lax.axis_index("s")
    core = lax.axis_index("c")
    lanes = lax.iota(jnp.int32, _L)

    @pl.when(core == 0)
    def _core0():
        _row_work(fmagn_a, fmagn_n, sls_a, sls_n, tab_a, tab_n,
                  stage_hbm, fm_v, sl_v, rows_v, res_v, sem, sem2, w, lanes)
        plsc.subcore_barrier()

        @pl.when(w == 0)
        def _final():
            _combine(ld_hbm, out_hbm, stage_hbm, res_v, tmp_v, tmp2_v, ld_v,
                     cmb_v, lanes)


def _row_work(fmagn_a, fmagn_n, sls_a, sls_n, tab_a, tab_n,
              stage_hbm, fm_v, sl_v, rows_v, res_v, sem, sem2, w, lanes):
    @pl.when(w < _B)
    def _():
        pltpu.async_copy(fmagn_a.at[w], fm_v, sem)
        pltpu.async_copy(sls_a.at[w], sl_v, sem2)

    @pl.when(w >= _B)
    def _():
        pltpu.async_copy(fmagn_n.at[w - _B], fm_v, sem)
        pltpu.async_copy(sls_n.at[w - _B], sl_v, sem2)

    pltpu.make_async_copy(fmagn_a.at[0], fm_v, sem).wait()

    neg_inf = jnp.full((_L,), -jnp.inf, jnp.float32)
    zero_i = jnp.zeros((_L,), jnp.int32)

    def insert(stack, v, ix):
        t1, i1, t2, i2, t3, i3 = stack
        c1 = v > t1
        nt1 = jnp.where(c1, v, t1)
        ni1 = jnp.where(c1, ix, i1)
        dv = jnp.where(c1, t1, v)
        di = jnp.where(c1, i1, ix)
        c2 = dv > t2
        nt2 = jnp.where(c2, dv, t2)
        ni2 = jnp.where(c2, di, i2)
        dv2 = jnp.where(c2, t2, dv)
        di2 = jnp.where(c2, i2, di)
        c3 = dv2 > t3
        nt3 = jnp.where(c3, dv2, t3)
        ni3 = jnp.where(c3, di2, i3)
        return (nt1, ni1, nt2, ni2, nt3, ni3)

    def insert_lex(stack, v, ix):
        t1, i1, t2, i2, t3, i3 = stack
        c1 = (v > t1) | ((v == t1) & (ix < i1))
        nt1 = jnp.where(c1, v, t1)
        ni1 = jnp.where(c1, ix, i1)
        dv = jnp.where(c1, t1, v)
        di = jnp.where(c1, i1, ix)
        c2 = (dv > t2) | ((dv == t2) & (di < i2))
        nt2 = jnp.where(c2, dv, t2)
        ni2 = jnp.where(c2, di, i2)
        dv2 = jnp.where(c2, t2, dv)
        di2 = jnp.where(c2, i2, di)
        c3 = (dv2 > t3) | ((dv2 == t3) & (di2 < i3))
        nt3 = jnp.where(c3, dv2, t3)
        ni3 = jnp.where(c3, di2, i3)
        return (nt1, ni1, nt2, ni2, nt3, ni3)

    empty = (neg_inf, zero_i, neg_inf, zero_i, neg_inf, zero_i)

    def topk_step(j, carry):
        sa = carry[:6]
        sb = carry[6:]
        va = fm_v[pl.ds((2 * j) * _L, _L)]
        vb = fm_v[pl.ds((2 * j + 1) * _L, _L)]
        ixa = lanes + (2 * j) * _L
        ixb = lanes + (2 * j + 1) * _L
        return insert(sa, va, ixa) + insert(sb, vb, ixb)

    st = lax.fori_loop(0, _NCHUNK // 2, topk_step, empty + empty)
    t1, i1, t2, i2, t3, i3 = st[:6]
    sb = st[6:]
    for lvl in range(3):
        t1, i1, t2, i2, t3, i3 = insert_lex(
            (t1, i1, t2, i2, t3, i3), sb[2 * lvl], sb[2 * lvl + 1])

    sel = []
    for _r in range(_K):
        m = jnp.max(t1)
        s = jnp.min(jnp.where(t1 == m, i1, _T))
        rm = i1 == s
        t1 = jnp.where(rm, t2, t1)
        i1 = jnp.where(rm, i2, i1)
        t2 = jnp.where(rm, t3, t2)
        i2 = jnp.where(rm, i3, i2)
        t3 = jnp.where(rm, neg_inf, t3)
        sel.append(s)

    selv = jnp.where(lanes == 1, sel[1],
                     jnp.where(lanes == 2, sel[2], sel[0]))
    pltpu.make_async_copy(sls_a.at[0], sl_v, sem2).wait()
    g = plsc.load_gather(sl_v, [selv])
    vls = jnp.sum(jnp.where(lanes < _K, g, 0.0)) * (1.0 / _K)

    b = jnp.where(w < _B, w, w - _B)
    row_iv = b * _T + selv

    @pl.when(w < _B)
    def _():
        pltpu.async_copy(tab_a.at[row_iv], rows_v, sem).wait()

    @pl.when(w >= _B)
    def _():
        pltpu.async_copy(tab_n.at[row_iv], rows_v, sem).wait()

    def ssq_step(c, accs):
        acc0, acc1 = accs
        d0 = pl.ds((2 * c) * _L, _L)
        d1 = pl.ds((2 * c + 1) * _L, _L)
        a0 = (rows_v[0, d0] + rows_v[1, d0] + rows_v[2, d0]) * (1.0 / 3.0)
        a1 = (rows_v[0, d1] + rows_v[1, d1] + rows_v[2, d1]) * (1.0 / 3.0)
        return (acc0 + a0 * a0, acc1 + a1 * a1)

    z = jnp.zeros((_L,), jnp.float32)
    acc0, acc1 = lax.fori_loop(0, _F // (2 * _L), ssq_step, (z, z))
    ssq = jnp.sum(acc0 + acc1)

    res_v[...] = jnp.where(lanes == 0, ssq, jnp.where(lanes == 1, vls, 0.0))
    pltpu.sync_copy(res_v, stage_hbm.at[w])


def _combine(ld_hbm, out_hbm, stage_hbm, res_v, tmp_v, tmp2_v, ld_v,
             cmb_v, lanes):
    if True:
        pltpu.sync_copy(stage_hbm, cmb_v)
        pltpu.sync_copy(ld_hbm, ld_v)
        zeros = jnp.zeros((_L,), jnp.int32)
        ssqv = plsc.load_gather(cmb_v, [lanes, zeros])
        vlsv = plsc.load_gather(cmb_v, [lanes, zeros + 1])

        sq = _sqrt(ssqv)
        tmp_v[...] = sq
        tmp2_v[...] = vlsv
        rot = (lanes + _B) & (_L - 1)
        sq_shift = plsc.load_gather(tmp_v, [rot])
        vc = plsc.load_gather(tmp2_v, [rot])

        la = jnp.abs(_MARGIN - sq)
        terms = (la + sq_shift) * (la + sq_shift)
        loss_rtfm = jnp.sum(jnp.where(lanes < _B, terms, 0.0)) * (1.0 / _B)

        ld = ld_v[...]
        ln_p = jnp.maximum(_ln(vc, lanes), -100.0)
        ln_1mp = jnp.maximum(_ln(1.0 - vc, lanes), -100.0)
        bce = -(ld * ln_p + (1.0 - ld) * ln_1mp)
        loss_vls = jnp.sum(bce) * (1.0 / _NROW)

        res_v[...] = jnp.where(lanes == 0, _ALPHA * loss_rtfm,
                               jnp.where(lanes == 1, loss_vls, 0.0))
        pltpu.sync_copy(res_v, out_hbm)


_sc_call = pl.kernel(
    _sc_body,
    out_type=[jax.ShapeDtypeStruct((_L,), jnp.float32),
              jax.ShapeDtypeStruct((_NROW, _L), jnp.float32)],
    mesh=plsc.VectorSubcoreMesh(core_axis_name="c", subcore_axis_name="s"),
    scratch_types=[
        pltpu.VMEM((_T,), jnp.float32),
        pltpu.VMEM((_T,), jnp.float32),
        pltpu.VMEM((_L, _F), jnp.float32),
        pltpu.VMEM((_L,), jnp.float32),
        pltpu.VMEM((_L,), jnp.float32),
        pltpu.VMEM((_L,), jnp.float32),
        pltpu.VMEM((_L,), jnp.float32),
        pltpu.VMEM((_NROW, _L), jnp.float32),
        pltpu.SemaphoreType.DMA,
        pltpu.SemaphoreType.DMA,
    ],
    compiler_params=pltpu.CompilerParams(needs_layout_passes=False),
)


def kernel(abnr_fmagn, norm_fmagn, abnr_feats, norm_feats, abnr_sls, norm_sls, ldata):
    tab_a = abnr_feats.reshape(_B * _T, _F)
    tab_n = norm_feats.reshape(_B * _T, _F)
    out, _stage = _sc_call(abnr_fmagn, norm_fmagn, abnr_sls, norm_sls,
                           tab_a, tab_n, ldata)
    return (out[0], out[1])
